# baseline (speedup 1.0000x reference)
import jax
import jax.numpy as jnp
from jax import lax
from jax.experimental import pallas as pl
from jax.experimental.pallas import tpu as pltpu
from jax.experimental.pallas import tpu_sc as plsc

N = 10000
NPAD = 10240
E = 320000
F = 128
NPADE = 240
ETOT = E + N + NPADE
EPT = ETOT // 32
C = 48
NCHUNK = EPT // C
NPT = NPAD // 16

_RB = 1024


def _tc_body(x_ref, w_ref, as_ref, ad_ref, h_ref, asv_ref, adv_ref):
    h = jnp.dot(x_ref[...], w_ref[...], preferred_element_type=jnp.float32)
    h_ref[...] = h
    asv_ref[...] = jnp.sum(h * as_ref[...], axis=1).reshape(_RB // 128, 128)
    adv_ref[...] = jnp.sum(h * ad_ref[...], axis=1).reshape(_RB // 128, 128)


def _tc_stage(x_pad, W, att_src, att_dst):
    return pl.pallas_call(
        _tc_body,
        grid=(NPAD // _RB,),
        in_specs=[
            pl.BlockSpec((_RB, F), lambda i: (i, 0)),
            pl.BlockSpec((F, F), lambda i: (0, 0)),
            pl.BlockSpec((1, F), lambda i: (0, 0)),
            pl.BlockSpec((1, F), lambda i: (0, 0)),
        ],
        out_specs=[
            pl.BlockSpec((_RB, F), lambda i: (i, 0)),
            pl.BlockSpec((_RB // 128, 128), lambda i: (i, 0)),
            pl.BlockSpec((_RB // 128, 128), lambda i: (i, 0)),
        ],
        out_shape=[
            jax.ShapeDtypeStruct((NPAD, F), jnp.float32),
            jax.ShapeDtypeStruct((NPAD // 128, 128), jnp.float32),
            jax.ShapeDtypeStruct((NPAD // 128, 128), jnp.float32),
        ],
    )(x_pad, W, att_src[None, :], att_dst[None, :])


def _comb_body(acc_ref, den_ref, b_ref, o_ref):
    a = acc_ref[0] + acc_ref[1]
    d = den_ref[0] + den_ref[1]
    r = 1.0 / (d + jnp.float32(1e-16))
    o_ref[...] = a * r[:, None] + b_ref[...]


def _comb_stage(acc2, den2, bias):
    return pl.pallas_call(
        _comb_body,
        grid=(NPAD // _RB,),
        in_specs=[
            pl.BlockSpec((2, _RB, F), lambda i: (0, i, 0)),
            pl.BlockSpec((2, _RB), lambda i: (0, i)),
            pl.BlockSpec((1, F), lambda i: (0, 0)),
        ],
        out_specs=pl.BlockSpec((_RB, F), lambda i: (i, 0)),
        out_shape=jax.ShapeDtypeStruct((NPAD, F), jnp.float32),
    )(acc2, den2, bias[None, :])


def _make_sc_kernel():
    mesh = plsc.VectorSubcoreMesh(core_axis_name="c", subcore_axis_name="s")

    def body(h_hbm, a_s_hbm, a_d_hbm, src_hbm, dst_hbm, acc_out, den_out,
             asl, adl, srcv6, dstv6, sv6, rows0, rows1, rows2, zbuf,
             acc_sh, den_sh, isem, gsem0, gsem1, gsem2,
             ssem0, ssem1, ssem2):
        cid = lax.axis_index("c")
        sid = lax.axis_index("s")
        zero16 = jnp.zeros((16,), jnp.float32)
        rowsL = [rows0, rows1, rows2]
        gsemL = [gsem0, gsem1, gsem2]
        ssemL = [ssem0, ssem1, ssem2]

        pltpu.sync_copy(a_s_hbm, asl)
        pltpu.sync_copy(a_d_hbm, adl)

        def _zero_rows(i, carry):
            for t in range(F // 16):
                rows0[i, pl.ds(t * 16, 16)] = zero16
            return carry
        lax.fori_loop(0, C, _zero_rows, 0)

        def _zero_z(i, carry):
            zbuf[pl.ds(i * 16, 16)] = zero16
            return carry
        lax.fori_loop(0, NPT // 16, _zero_z, 0)

        n0 = pl.multiple_of(sid * NPT, 64)
        for off in range(0, NPT - C, C):
            pltpu.sync_copy(rows0, acc_sh.at[pl.ds(n0 + off, C)])
        rem = NPT % C
        pltpu.sync_copy(rows0.at[pl.ds(0, rem)],
                        acc_sh.at[pl.ds(n0 + NPT - rem, rem)])
        pltpu.sync_copy(zbuf, den_sh.at[pl.ds(n0, NPT)])
        plsc.subcore_barrier()

        ebase = pl.multiple_of((cid * 16 + sid) * EPT, 8)

        def idx_start(g, s6):
            base = pl.multiple_of(ebase + g * C, 8)
            pltpu.async_copy(src_hbm.at[pl.ds(base, C)], srcv6.at[s6], isem)
            pltpu.async_copy(dst_hbm.at[pl.ds(base, C)], dstv6.at[s6], isem)

        def idx_wait(s6):
            pltpu.make_async_copy(
                src_hbm.at[pl.ds(0, C)], srcv6.at[s6], isem).wait()
            pltpu.make_async_copy(
                dst_hbm.at[pl.ds(0, C)], dstv6.at[s6], isem).wait()

        def scomp(s6):
            for grp in range(C // 16):
                sl = pl.ds(grp * 16, 16)
                s16 = srcv6[s6, sl]
                d16 = dstv6[s6, sl]
                a1 = plsc.load_gather(asl, [s16])
                a2 = plsc.load_gather(adl, [d16])
                e = a1 + a2
                e = jnp.where(e >= 0.0, e, e * jnp.float32(0.2))
                sv6[s6, sl] = jnp.exp(e)

        def gath_start(b, s6):
            pltpu.async_copy(h_hbm.at[srcv6.at[s6]], rowsL[b], gsemL[b])

        def gath_wait(b, s6):
            pltpu.make_async_copy(
                h_hbm.at[srcv6.at[s6]], rowsL[b], gsemL[b]).wait()

        def scale(b, s6):
            rows = rowsL[b]

            @plsc.parallel_loop(0, C // 16, unroll=C // 16)
            def sbody(grp):
                s16 = sv6[s6, pl.ds(grp * 16, 16)]
                for l in range(16):
                    j = grp * 16 + l
                    ss = s16[l]
                    for t in range(F // 16):
                        tsl = pl.ds(t * 16, 16)
                        rows[j, tsl] = rows[j, tsl] * ss

        def scat_start(b, s6):
            pltpu.async_copy(
                rowsL[b], acc_sh.at[dstv6.at[s6]], ssemL[b], add=True)
            pltpu.async_copy(
                sv6.at[s6], den_sh.at[dstv6.at[s6]], ssemL[b], add=True)

        def scat_wait(b, s6):
            pltpu.make_async_copy(
                rowsL[b], acc_sh.at[dstv6.at[s6]], ssemL[b]).wait()
            pltpu.make_async_copy(
                sv6.at[s6], den_sh.at[dstv6.at[s6]], ssemL[b]).wait()

        def slot(g, b, s6, prep, wait_prev):
            gath_wait(b, s6)
            if prep:
                idx_start(g + 2, (s6 + 2) % 6)
            scale(b, s6)
            scat_start(b, s6)
            if wait_prev:
                scat_wait((b + 2) % 3, (s6 + 5) % 6)
            if prep:
                s6p = (s6 + 2) % 6
                idx_wait(s6p)
                scomp(s6p)
                gath_start((b + 2) % 3, s6p)

        idx_start(0, 0)
        idx_wait(0)
        scomp(0)
        gath_start(0, 0)
        idx_start(1, 1)
        idx_wait(1)
        scomp(1)
        gath_start(1, 1)
        slot(0, 0, 0, True, False)
        slot(1, 1, 1, True, True)

        NSIX = (NCHUNK - 2) // 6

        def six(i, carry):
            g0 = 2 + i * 6
            for b6 in range(6):
                slot(g0 + b6, (2 + b6) % 3, (2 + b6) % 6, True, True)
            return carry
        lax.fori_loop(0, NSIX, six, 0)

        for g in range(2 + 6 * NSIX, NCHUNK):
            slot(g, g % 3, g % 6, g + 2 < NCHUNK, True)
        scat_wait((NCHUNK - 1) % 3, (NCHUNK - 1) % 6)

        plsc.subcore_barrier()

        row0 = pl.multiple_of(cid * NPAD + sid * NPT, 64)
        pltpu.sync_copy(acc_sh.at[pl.ds(n0, NPT)],
                        acc_out.at[pl.ds(row0, NPT)])
        pltpu.sync_copy(den_sh.at[pl.ds(n0, NPT)],
                        den_out.at[pl.ds(row0, NPT)])

    return pl.kernel(
        body,
        out_type=(jax.ShapeDtypeStruct((2 * NPAD, F), jnp.float32),
                  jax.ShapeDtypeStruct((2 * NPAD,), jnp.float32)),
        mesh=mesh,
        compiler_params=pltpu.CompilerParams(
            needs_layout_passes=False, use_tc_tiling_on_sc=False),
        scratch_types=[
            pltpu.VMEM((NPAD,), jnp.float32),
            pltpu.VMEM((NPAD,), jnp.float32),
            pltpu.VMEM((6, C), jnp.int32),
            pltpu.VMEM((6, C), jnp.int32),
            pltpu.VMEM((6, C), jnp.float32),
            pltpu.VMEM((C, F), jnp.float32),
            pltpu.VMEM((C, F), jnp.float32),
            pltpu.VMEM((C, F), jnp.float32),
            pltpu.VMEM((NPT,), jnp.float32),
            pltpu.VMEM_SHARED((NPAD, F), jnp.float32),
            pltpu.VMEM_SHARED((NPAD,), jnp.float32),
            pltpu.SemaphoreType.DMA,
            pltpu.SemaphoreType.DMA,
            pltpu.SemaphoreType.DMA,
            pltpu.SemaphoreType.DMA,
            pltpu.SemaphoreType.DMA,
            pltpu.SemaphoreType.DMA,
            pltpu.SemaphoreType.DMA,
        ],
    )


def kernel(x, edge_index, W, att_src, att_dst, bias):
    h, asv, adv = _tc_stage(x, W, att_src, att_dst)
    a_s = asv.reshape(NPAD)
    a_d = adv.reshape(NPAD)

    ids = edge_index.astype(jnp.int32)
    loops = jnp.arange(N, dtype=jnp.int32)
    padl = N + jnp.arange(NPADE, dtype=jnp.int32)
    src_all = jnp.concatenate([ids[0], loops, padl])
    dst_all = jnp.concatenate([ids[1], loops, padl])

    acc2, den2 = _make_sc_kernel()(h, a_s, a_d, src_all, dst_all)
    out = _comb_stage(acc2.reshape(2, NPAD, F), den2.reshape(2, NPAD), bias)
    return out[:N]


# speedup vs baseline: 1.0967x; 1.0967x over previous
import jax
import jax.numpy as jnp
from jax import lax
from jax.experimental import pallas as pl
from jax.experimental.pallas import tpu as pltpu
from jax.experimental.pallas import tpu_sc as plsc

N = 10000
NPAD = 10240
E = 320000
F = 128
NPADE = 240
ETOT = E + N + NPADE
EPT = ETOT // 32
C = 48
NCHUNK = EPT // C
NPT = NPAD // 16

_RB = 1024


def _tc_body(x_ref, w_ref, as_ref, ad_ref, h_ref, asv_ref, adv_ref):
    h = jnp.dot(x_ref[...], w_ref[...], preferred_element_type=jnp.float32)
    h_ref[...] = h
    asv_ref[...] = jnp.sum(h * as_ref[...], axis=1).reshape(_RB // 128, 128)
    adv_ref[...] = jnp.sum(h * ad_ref[...], axis=1).reshape(_RB // 128, 128)


def _tc_stage(x_pad, W, att_src, att_dst):
    return pl.pallas_call(
        _tc_body,
        grid=(NPAD // _RB,),
        in_specs=[
            pl.BlockSpec((_RB, F), lambda i: (i, 0)),
            pl.BlockSpec((F, F), lambda i: (0, 0)),
            pl.BlockSpec((1, F), lambda i: (0, 0)),
            pl.BlockSpec((1, F), lambda i: (0, 0)),
        ],
        out_specs=[
            pl.BlockSpec((_RB, F), lambda i: (i, 0)),
            pl.BlockSpec((_RB // 128, 128), lambda i: (i, 0)),
            pl.BlockSpec((_RB // 128, 128), lambda i: (i, 0)),
        ],
        out_shape=[
            jax.ShapeDtypeStruct((NPAD, F), jnp.float32),
            jax.ShapeDtypeStruct((NPAD // 128, 128), jnp.float32),
            jax.ShapeDtypeStruct((NPAD // 128, 128), jnp.float32),
        ],
    )(x_pad, W, att_src[None, :], att_dst[None, :])


def _comb_body(acc_ref, den_ref, b_ref, o_ref):
    a = acc_ref[0] + acc_ref[1]
    d = den_ref[0] + den_ref[1]
    r = 1.0 / (d + jnp.float32(1e-16))
    o_ref[...] = a * r[:, None] + b_ref[...]


def _comb_stage(acc2, den2, bias):
    return pl.pallas_call(
        _comb_body,
        grid=(NPAD // _RB,),
        in_specs=[
            pl.BlockSpec((2, _RB, F), lambda i: (0, i, 0)),
            pl.BlockSpec((2, _RB), lambda i: (0, i)),
            pl.BlockSpec((1, F), lambda i: (0, 0)),
        ],
        out_specs=pl.BlockSpec((_RB, F), lambda i: (i, 0)),
        out_shape=jax.ShapeDtypeStruct((NPAD, F), jnp.float32),
    )(acc2, den2, bias[None, :])


def _make_sc_kernel():
    mesh = plsc.VectorSubcoreMesh(core_axis_name="c", subcore_axis_name="s")

    def body(h_hbm, a_s_hbm, a_d_hbm, src_hbm, dst_hbm, acc_out, den_out,
             asl, adl, srcv6, dstv6, sv6, rows0, rows1, rows2, zbuf,
             acc_sh, den_sh, isem, gsem0, gsem1, gsem2,
             ssem0, ssem1, ssem2):
        cid = lax.axis_index("c")
        sid = lax.axis_index("s")
        zero16 = jnp.zeros((16,), jnp.float32)
        rowsL = [rows0, rows1, rows2]
        gsemL = [gsem0, gsem1, gsem2]
        ssemL = [ssem0, ssem1, ssem2]

        pltpu.sync_copy(a_s_hbm, asl)
        pltpu.sync_copy(a_d_hbm, adl)

        def _zero_rows(i, carry):
            for t in range(F // 16):
                rows0[i, pl.ds(t * 16, 16)] = zero16
            return carry
        lax.fori_loop(0, C, _zero_rows, 0)

        def _zero_z(i, carry):
            zbuf[pl.ds(i * 16, 16)] = zero16
            return carry
        lax.fori_loop(0, NPT // 16, _zero_z, 0)

        n0 = pl.multiple_of(sid * NPT, 64)
        for off in range(0, NPT - C, C):
            pltpu.sync_copy(rows0, acc_sh.at[pl.ds(n0 + off, C)])
        rem = NPT % C
        pltpu.sync_copy(rows0.at[pl.ds(0, rem)],
                        acc_sh.at[pl.ds(n0 + NPT - rem, rem)])
        pltpu.sync_copy(zbuf, den_sh.at[pl.ds(n0, NPT)])
        plsc.subcore_barrier()

        ebase = pl.multiple_of((cid * 16 + sid) * EPT, 8)

        def idx_start(g, s6):
            base = pl.multiple_of(ebase + g * C, 8)
            pltpu.async_copy(src_hbm.at[pl.ds(base, C)], srcv6.at[s6], isem)
            pltpu.async_copy(dst_hbm.at[pl.ds(base, C)], dstv6.at[s6], isem)

        def idx_wait(s6):
            pltpu.make_async_copy(
                src_hbm.at[pl.ds(0, C)], srcv6.at[s6], isem).wait()
            pltpu.make_async_copy(
                dst_hbm.at[pl.ds(0, C)], dstv6.at[s6], isem).wait()

        def scomp(s6):
            for grp in range(C // 16):
                sl = pl.ds(grp * 16, 16)
                s16 = srcv6[s6, sl]
                d16 = dstv6[s6, sl]
                a1 = plsc.load_gather(asl, [s16])
                a2 = plsc.load_gather(adl, [d16])
                e = a1 + a2
                e = jnp.where(e >= 0.0, e, e * jnp.float32(0.2))
                sv6[s6, sl] = jnp.exp(e)

        def gath_start(b, s6):
            pltpu.async_copy(h_hbm.at[srcv6.at[s6]], rowsL[b], gsemL[b])

        def gath_wait(b, s6):
            pltpu.make_async_copy(
                h_hbm.at[srcv6.at[s6]], rowsL[b], gsemL[b]).wait()

        def scale(b, s6):
            rows = rowsL[b]

            def sbody(grp, carry):
                s16 = sv6[s6, pl.ds(grp * 16, 16)]
                for l in range(16):
                    j = grp * 16 + l
                    ss = s16[l]
                    for t in range(F // 16):
                        tsl = pl.ds(t * 16, 16)
                        rows[j, tsl] = rows[j, tsl] * ss
                return carry
            lax.fori_loop(0, C // 16, sbody, 0)

        def scat_start(b, s6):
            pltpu.async_copy(
                rowsL[b], acc_sh.at[dstv6.at[s6]], ssemL[b], add=True)
            pltpu.async_copy(
                sv6.at[s6], den_sh.at[dstv6.at[s6]], ssemL[b], add=True)

        def scat_wait(b, s6):
            pltpu.make_async_copy(
                rowsL[b], acc_sh.at[dstv6.at[s6]], ssemL[b]).wait()
            pltpu.make_async_copy(
                sv6.at[s6], den_sh.at[dstv6.at[s6]], ssemL[b]).wait()

        def slot(g, b, s6, prep, wait_prev):
            gath_wait(b, s6)
            if prep:
                idx_start(g + 2, (s6 + 2) % 6)
            scale(b, s6)
            scat_start(b, s6)
            if wait_prev:
                scat_wait((b + 2) % 3, (s6 + 5) % 6)
            if prep:
                s6p = (s6 + 2) % 6
                idx_wait(s6p)
                scomp(s6p)
                gath_start((b + 2) % 3, s6p)

        idx_start(0, 0)
        idx_wait(0)
        scomp(0)
        gath_start(0, 0)
        idx_start(1, 1)
        idx_wait(1)
        scomp(1)
        gath_start(1, 1)
        slot(0, 0, 0, True, False)
        slot(1, 1, 1, True, True)

        NSIX = (NCHUNK - 2) // 6

        def six(i, carry):
            g0 = 2 + i * 6
            for b6 in range(6):
                slot(g0 + b6, (2 + b6) % 3, (2 + b6) % 6, True, True)
            return carry
        lax.fori_loop(0, NSIX, six, 0)

        for g in range(2 + 6 * NSIX, NCHUNK):
            slot(g, g % 3, g % 6, g + 2 < NCHUNK, True)
        scat_wait((NCHUNK - 1) % 3, (NCHUNK - 1) % 6)

        plsc.subcore_barrier()

        row0 = pl.multiple_of(cid * NPAD + sid * NPT, 64)
        pltpu.sync_copy(acc_sh.at[pl.ds(n0, NPT)],
                        acc_out.at[pl.ds(row0, NPT)])
        pltpu.sync_copy(den_sh.at[pl.ds(n0, NPT)],
                        den_out.at[pl.ds(row0, NPT)])

    return pl.kernel(
        body,
        out_type=(jax.ShapeDtypeStruct((2 * NPAD, F), jnp.float32),
                  jax.ShapeDtypeStruct((2 * NPAD,), jnp.float32)),
        mesh=mesh,
        compiler_params=pltpu.CompilerParams(
            needs_layout_passes=False, use_tc_tiling_on_sc=False),
        scratch_types=[
            pltpu.VMEM((NPAD,), jnp.float32),
            pltpu.VMEM((NPAD,), jnp.float32),
            pltpu.VMEM((6, C), jnp.int32),
            pltpu.VMEM((6, C), jnp.int32),
            pltpu.VMEM((6, C), jnp.float32),
            pltpu.VMEM((C, F), jnp.float32),
            pltpu.VMEM((C, F), jnp.float32),
            pltpu.VMEM((C, F), jnp.float32),
            pltpu.VMEM((NPT,), jnp.float32),
            pltpu.VMEM_SHARED((NPAD, F), jnp.float32),
            pltpu.VMEM_SHARED((NPAD,), jnp.float32),
            pltpu.SemaphoreType.DMA,
            pltpu.SemaphoreType.DMA,
            pltpu.SemaphoreType.DMA,
            pltpu.SemaphoreType.DMA,
            pltpu.SemaphoreType.DMA,
            pltpu.SemaphoreType.DMA,
            pltpu.SemaphoreType.DMA,
        ],
    )


def kernel(x, edge_index, W, att_src, att_dst, bias):
    h, asv, adv = _tc_stage(x, W, att_src, att_dst)
    a_s = asv.reshape(NPAD)
    a_d = adv.reshape(NPAD)

    ids = edge_index.astype(jnp.int32)
    loops = jnp.arange(N, dtype=jnp.int32)
    padl = N + jnp.arange(NPADE, dtype=jnp.int32)
    src_all = jnp.concatenate([ids[0], loops, padl])
    dst_all = jnp.concatenate([ids[1], loops, padl])

    acc2, den2 = _make_sc_kernel()(h, a_s, a_d, src_all, dst_all)
    out = _comb_stage(acc2.reshape(2, NPAD, F), den2.reshape(2, NPAD), bias)
    return out[:N]


# speedup vs baseline: 1.0996x; 1.0027x over previous
import jax
import jax.numpy as jnp
from jax import lax
from jax.experimental import pallas as pl
from jax.experimental.pallas import tpu as pltpu
from jax.experimental.pallas import tpu_sc as plsc

N = 10000
NPAD = 10240
E = 320000
F = 128
NPADE = 240
ETOT = E + N + NPADE
EPT = ETOT // 32
C = 48
NCHUNK = EPT // C
NPT = NPAD // 16

_RB = 1024


def _tc_body(x_ref, w_ref, as_ref, ad_ref, h_ref, asv_ref, adv_ref):
    h = jnp.dot(x_ref[...], w_ref[...], preferred_element_type=jnp.float32)
    h_ref[...] = h
    asv_ref[...] = jnp.sum(h * as_ref[...], axis=1).reshape(_RB // 128, 128)
    adv_ref[...] = jnp.sum(h * ad_ref[...], axis=1).reshape(_RB // 128, 128)


def _tc_stage(x_pad, W, att_src, att_dst):
    return pl.pallas_call(
        _tc_body,
        grid=(NPAD // _RB,),
        in_specs=[
            pl.BlockSpec((_RB, F), lambda i: (i, 0)),
            pl.BlockSpec((F, F), lambda i: (0, 0)),
            pl.BlockSpec((1, F), lambda i: (0, 0)),
            pl.BlockSpec((1, F), lambda i: (0, 0)),
        ],
        out_specs=[
            pl.BlockSpec((_RB, F), lambda i: (i, 0)),
            pl.BlockSpec((_RB // 128, 128), lambda i: (i, 0)),
            pl.BlockSpec((_RB // 128, 128), lambda i: (i, 0)),
        ],
        out_shape=[
            jax.ShapeDtypeStruct((NPAD, F), jnp.float32),
            jax.ShapeDtypeStruct((NPAD // 128, 128), jnp.float32),
            jax.ShapeDtypeStruct((NPAD // 128, 128), jnp.float32),
        ],
    )(x_pad, W, att_src[None, :], att_dst[None, :])


def _comb_body(acc_ref, den_ref, b_ref, o_ref):
    a = acc_ref[0] + acc_ref[1]
    d = den_ref[0] + den_ref[1]
    r = 1.0 / (d + jnp.float32(1e-16))
    o_ref[...] = a * r[:, None] + b_ref[...]


def _comb_stage(acc2, den2, bias):
    return pl.pallas_call(
        _comb_body,
        grid=(NPAD // _RB,),
        in_specs=[
            pl.BlockSpec((2, _RB, F), lambda i: (0, i, 0)),
            pl.BlockSpec((2, _RB), lambda i: (0, i)),
            pl.BlockSpec((1, F), lambda i: (0, 0)),
        ],
        out_specs=pl.BlockSpec((_RB, F), lambda i: (i, 0)),
        out_shape=jax.ShapeDtypeStruct((NPAD, F), jnp.float32),
    )(acc2, den2, bias[None, :])


def _make_sc_kernel():
    mesh = plsc.VectorSubcoreMesh(core_axis_name="c", subcore_axis_name="s")

    def body(h_hbm, a_s_hbm, a_d_hbm, src_hbm, dst_hbm, acc_out, den_out,
             asl, adl, srcv6, dstv6, sv6, rows0, rows1, rows2, zbuf,
             acc_sh, den_sh, isem, gsem0, gsem1, gsem2,
             ssem0, ssem1, ssem2):
        cid = lax.axis_index("c")
        sid = lax.axis_index("s")
        zero16 = jnp.zeros((16,), jnp.float32)
        rowsL = [rows0, rows1, rows2]
        gsemL = [gsem0, gsem1, gsem2]
        ssemL = [ssem0, ssem1, ssem2]

        pltpu.sync_copy(a_s_hbm, asl)
        pltpu.sync_copy(a_d_hbm, adl)

        def _zero_rows(i, carry):
            for t in range(F // 16):
                rows0[i, pl.ds(t * 16, 16)] = zero16
            return carry
        lax.fori_loop(0, C, _zero_rows, 0)

        def _zero_z(i, carry):
            zbuf[pl.ds(i * 16, 16)] = zero16
            return carry
        lax.fori_loop(0, NPT // 16, _zero_z, 0)

        n0 = pl.multiple_of(sid * NPT, 64)
        for off in range(0, NPT - C, C):
            pltpu.sync_copy(rows0, acc_sh.at[pl.ds(n0 + off, C)])
        rem = NPT % C
        pltpu.sync_copy(rows0.at[pl.ds(0, rem)],
                        acc_sh.at[pl.ds(n0 + NPT - rem, rem)])
        pltpu.sync_copy(zbuf, den_sh.at[pl.ds(n0, NPT)])
        plsc.subcore_barrier()

        ebase = pl.multiple_of((cid * 16 + sid) * EPT, 8)

        def idx_start(g, s6):
            base = pl.multiple_of(ebase + g * C, 8)
            pltpu.async_copy(src_hbm.at[pl.ds(base, C)], srcv6.at[s6], isem)
            pltpu.async_copy(dst_hbm.at[pl.ds(base, C)], dstv6.at[s6], isem)

        def idx_wait(s6):
            pltpu.make_async_copy(
                src_hbm.at[pl.ds(0, C)], srcv6.at[s6], isem).wait()
            pltpu.make_async_copy(
                dst_hbm.at[pl.ds(0, C)], dstv6.at[s6], isem).wait()

        def scomp(s6):
            for grp in range(C // 16):
                sl = pl.ds(grp * 16, 16)
                s16 = srcv6[s6, sl]
                d16 = dstv6[s6, sl]
                a1 = plsc.load_gather(asl, [s16])
                a2 = plsc.load_gather(adl, [d16])
                e = a1 + a2
                e = jnp.where(e >= 0.0, e, e * jnp.float32(0.2))
                sv6[s6, sl] = jnp.exp(e)

        def gath_start(b, s6):
            pltpu.async_copy(h_hbm.at[srcv6.at[s6]], rowsL[b], gsemL[b])

        def gath_wait(b, s6):
            pltpu.make_async_copy(
                h_hbm.at[srcv6.at[s6]], rowsL[b], gsemL[b]).wait()

        def scale(b, s6):
            rows = rowsL[b]

            @plsc.parallel_loop(0, C // 16)
            def sbody(grp):
                s16 = sv6[s6, pl.ds(grp * 16, 16)]
                for l in range(16):
                    j = grp * 16 + l
                    ss = s16[l]
                    for t in range(F // 16):
                        tsl = pl.ds(t * 16, 16)
                        rows[j, tsl] = rows[j, tsl] * ss

        def scat_start(b, s6):
            pltpu.async_copy(
                rowsL[b], acc_sh.at[dstv6.at[s6]], ssemL[b], add=True)
            pltpu.async_copy(
                sv6.at[s6], den_sh.at[dstv6.at[s6]], ssemL[b], add=True)

        def scat_wait(b, s6):
            pltpu.make_async_copy(
                rowsL[b], acc_sh.at[dstv6.at[s6]], ssemL[b]).wait()
            pltpu.make_async_copy(
                sv6.at[s6], den_sh.at[dstv6.at[s6]], ssemL[b]).wait()

        def slot(g, b, s6, prep, wait_prev):
            gath_wait(b, s6)
            if prep:
                idx_start(g + 2, (s6 + 2) % 6)
            scale(b, s6)
            scat_start(b, s6)
            if wait_prev:
                scat_wait((b + 2) % 3, (s6 + 5) % 6)
            if prep:
                s6p = (s6 + 2) % 6
                idx_wait(s6p)
                scomp(s6p)
                gath_start((b + 2) % 3, s6p)

        idx_start(0, 0)
        idx_wait(0)
        scomp(0)
        gath_start(0, 0)
        idx_start(1, 1)
        idx_wait(1)
        scomp(1)
        gath_start(1, 1)
        slot(0, 0, 0, True, False)
        slot(1, 1, 1, True, True)

        NSIX = (NCHUNK - 2) // 6

        def six(i, carry):
            g0 = 2 + i * 6
            for b6 in range(6):
                slot(g0 + b6, (2 + b6) % 3, (2 + b6) % 6, True, True)
            return carry
        lax.fori_loop(0, NSIX, six, 0)

        for g in range(2 + 6 * NSIX, NCHUNK):
            slot(g, g % 3, g % 6, g + 2 < NCHUNK, True)
        scat_wait((NCHUNK - 1) % 3, (NCHUNK - 1) % 6)

        plsc.subcore_barrier()

        row0 = pl.multiple_of(cid * NPAD + sid * NPT, 64)
        pltpu.sync_copy(acc_sh.at[pl.ds(n0, NPT)],
                        acc_out.at[pl.ds(row0, NPT)])
        pltpu.sync_copy(den_sh.at[pl.ds(n0, NPT)],
                        den_out.at[pl.ds(row0, NPT)])

    return pl.kernel(
        body,
        out_type=(jax.ShapeDtypeStruct((2 * NPAD, F), jnp.float32),
                  jax.ShapeDtypeStruct((2 * NPAD,), jnp.float32)),
        mesh=mesh,
        compiler_params=pltpu.CompilerParams(
            needs_layout_passes=False, use_tc_tiling_on_sc=False),
        scratch_types=[
            pltpu.VMEM((NPAD,), jnp.float32),
            pltpu.VMEM((NPAD,), jnp.float32),
            pltpu.VMEM((6, C), jnp.int32),
            pltpu.VMEM((6, C), jnp.int32),
            pltpu.VMEM((6, C), jnp.float32),
            pltpu.VMEM((C, F), jnp.float32),
            pltpu.VMEM((C, F), jnp.float32),
            pltpu.VMEM((C, F), jnp.float32),
            pltpu.VMEM((NPT,), jnp.float32),
            pltpu.VMEM_SHARED((NPAD, F), jnp.float32),
            pltpu.VMEM_SHARED((NPAD,), jnp.float32),
            pltpu.SemaphoreType.DMA,
            pltpu.SemaphoreType.DMA,
            pltpu.SemaphoreType.DMA,
            pltpu.SemaphoreType.DMA,
            pltpu.SemaphoreType.DMA,
            pltpu.SemaphoreType.DMA,
            pltpu.SemaphoreType.DMA,
        ],
    )


def kernel(x, edge_index, W, att_src, att_dst, bias):
    h, asv, adv = _tc_stage(x, W, att_src, att_dst)
    a_s = asv.reshape(NPAD)
    a_d = adv.reshape(NPAD)

    ids = edge_index.astype(jnp.int32)
    loops = jnp.arange(N, dtype=jnp.int32)
    padl = N + jnp.arange(NPADE, dtype=jnp.int32)
    src_all = jnp.concatenate([ids[0], loops, padl])
    dst_all = jnp.concatenate([ids[1], loops, padl])

    acc2, den2 = _make_sc_kernel()(h, a_s, a_d, src_all, dst_all)
    out = _comb_stage(acc2.reshape(2, NPAD, F), den2.reshape(2, NPAD), bias)
    return out[:N]
